# Initial kernel scaffold; baseline (speedup 1.0000x reference)
import jax
import jax.numpy as jnp
from jax.experimental import pallas as pl


def kernel(x, gw1, gb1, gw2, gb2, gw3, gb3, ew1, ew2, eb1, eb2):
    raise NotImplementedError("write your pallas kernel here")


# speedup vs baseline: 4.5009x; 4.5009x over previous
import jax
import jax.numpy as jnp
from jax.experimental import pallas as pl
from jax.experimental.pallas import tpu as pltpu

B, S, DIM, E, K = 2, 32, 512, 64, 2
H = 4 * DIM
T = B * S


_SQRT_HALF = 0.7071067811865476


def _gelu(t):
    return 0.5 * t * (1.0 + jax.lax.erf(t * _SQRT_HALF))


def _gate_kernel(x_ref, gw1_ref, gb1_ref, gw2_ref, gb2_ref, gw3_ref, gb3_ref,
                 i1_ref, i2_ref, v1_ref, v2_ref):
    hi = None
    xt = x_ref[...]
    g = _gelu(jnp.dot(xt, gw1_ref[...], precision=hi,
                      preferred_element_type=jnp.float32) + gb1_ref[0])
    g = _gelu(jnp.dot(g, gw2_ref[...], precision=hi,
                      preferred_element_type=jnp.float32) + gb2_ref[0])
    logits = jax.nn.sigmoid(jnp.dot(g, gw3_ref[...], precision=hi,
                                    preferred_element_type=jnp.float32) + gb3_ref[0])
    iota = jax.lax.broadcasted_iota(jnp.int32, (T, E), 1)
    v1 = jnp.max(logits, axis=1, keepdims=True)
    i1 = jnp.min(jnp.where(logits == v1, iota, E), axis=1, keepdims=True)
    masked = jnp.where(iota == i1, -jnp.inf, logits)
    v2 = jnp.max(masked, axis=1, keepdims=True)
    i2 = jnp.min(jnp.where(masked == v2, iota, E), axis=1, keepdims=True)
    s = v1 + v2
    i1_ref[...] = i1
    i2_ref[...] = i2
    v1_ref[...] = v1 / s
    v2_ref[...] = v2 / s


def _expert_kernel(i1_ref, i2_ref, v1_ref, v2_ref, x_ref,
                   ew1_ref, ew2_ref, eb1_ref, eb2_ref, out_ref):
    e = pl.program_id(0)
    hi = None
    w1 = ew1_ref[0]
    w2 = ew2_ref[0]
    h = _gelu(jax.lax.dot_general(x_ref[...], w1, (((1,), (1,)), ((), ())),
                                  precision=hi,
                                  preferred_element_type=jnp.float32)
              + eb1_ref[0])
    o = _gelu(jnp.dot(h, w2, precision=hi,
                      preferred_element_type=jnp.float32) + eb2_ref[0])
    scale = (jnp.where(i1_ref[...] == e, v1_ref[...], 0.0)
             + jnp.where(i2_ref[...] == e, v2_ref[...], 0.0))
    contrib = scale * o

    @pl.when(e == 0)
    def _init():
        out_ref[...] = contrib

    @pl.when(e != 0)
    def _acc():
        out_ref[...] += contrib


def kernel(x, gw1, gb1, gw2, gb2, gw3, gb3, ew1, ew2, eb1, eb2):
    xt = x.reshape(T, DIM)
    eb1r = eb1.reshape(E, 1, H)
    eb2r = eb2.reshape(E, 1, DIM)

    i1, i2, v1, v2 = pl.pallas_call(
        _gate_kernel,
        out_shape=(
            jax.ShapeDtypeStruct((T, 1), jnp.int32),
            jax.ShapeDtypeStruct((T, 1), jnp.int32),
            jax.ShapeDtypeStruct((T, 1), jnp.float32),
            jax.ShapeDtypeStruct((T, 1), jnp.float32),
        ),
    )(xt, gw1, gb1.reshape(1, H), gw2, gb2.reshape(1, H), gw3,
      gb3.reshape(1, E))

    out = pl.pallas_call(
        _expert_kernel,
        grid=(E,),
        in_specs=[
            pl.BlockSpec((T, 1), lambda e: (0, 0)),
            pl.BlockSpec((T, 1), lambda e: (0, 0)),
            pl.BlockSpec((T, 1), lambda e: (0, 0)),
            pl.BlockSpec((T, 1), lambda e: (0, 0)),
            pl.BlockSpec((T, DIM), lambda e: (0, 0)),
            pl.BlockSpec((1, H, DIM), lambda e: (e, 0, 0)),
            pl.BlockSpec((1, H, DIM), lambda e: (e, 0, 0)),
            pl.BlockSpec((1, 1, H), lambda e: (e, 0, 0)),
            pl.BlockSpec((1, 1, DIM), lambda e: (e, 0, 0)),
        ],
        out_specs=pl.BlockSpec((T, DIM), lambda e: (0, 0)),
        out_shape=jax.ShapeDtypeStruct((T, DIM), jnp.float32),
        compiler_params=pltpu.CompilerParams(
            dimension_semantics=("arbitrary",),
        ),
    )(i1, i2, v1, v2, xt, ew1, ew2, eb1r, eb2r)

    return out.reshape(B, S, DIM)


# speedup vs baseline: 4.5409x; 1.0089x over previous
import jax
import jax.numpy as jnp
from jax.experimental import pallas as pl
from jax.experimental.pallas import tpu as pltpu

B, S, DIM, E, K = 2, 32, 512, 64, 2
H = 4 * DIM
T = B * S


_SQRT_HALF = 0.7071067811865476


def _gelu(t):
    return 0.5 * t * (1.0 + jax.lax.erf(t * _SQRT_HALF))


def _gate_kernel(x_ref, gw1_ref, gb1_ref, gw2_ref, gb2_ref, gw3_ref, gb3_ref,
                 i1_ref, i2_ref, v1_ref, v2_ref):
    hi = None
    xt = x_ref[...]
    g = _gelu(jnp.dot(xt, gw1_ref[...], precision=hi,
                      preferred_element_type=jnp.float32) + gb1_ref[0])
    g = _gelu(jnp.dot(g, gw2_ref[...], precision=hi,
                      preferred_element_type=jnp.float32) + gb2_ref[0])
    logits = jax.nn.sigmoid(jnp.dot(g, gw3_ref[...], precision=hi,
                                    preferred_element_type=jnp.float32) + gb3_ref[0])
    iota = jax.lax.broadcasted_iota(jnp.int32, (T, E), 1)
    v1 = jnp.max(logits, axis=1, keepdims=True)
    i1 = jnp.min(jnp.where(logits == v1, iota, E), axis=1, keepdims=True)
    masked = jnp.where(iota == i1, -jnp.inf, logits)
    v2 = jnp.max(masked, axis=1, keepdims=True)
    i2 = jnp.min(jnp.where(masked == v2, iota, E), axis=1, keepdims=True)
    s = v1 + v2
    i1_ref[...] = i1
    i2_ref[...] = i2
    v1_ref[...] = v1 / s
    v2_ref[...] = v2 / s


EPB = 2


def _expert_kernel(i1_ref, i2_ref, v1_ref, v2_ref, x_ref,
                   ew1_ref, ew2_ref, eb1_ref, eb2_ref, out_ref):
    step = pl.program_id(0)
    hi = None
    w1 = ew1_ref[...].reshape(EPB * H, DIM)
    h = _gelu(jax.lax.dot_general(x_ref[...], w1, (((1,), (1,)), ((), ())),
                                  precision=hi,
                                  preferred_element_type=jnp.float32)
              + eb1_ref[...].reshape(1, EPB * H))
    acc = jnp.zeros((T, DIM), jnp.float32)
    for j in range(EPB):
        e = step * EPB + j
        o = _gelu(jnp.dot(h[:, j * H:(j + 1) * H], ew2_ref[j], precision=hi,
                          preferred_element_type=jnp.float32) + eb2_ref[j])
        scale = (jnp.where(i1_ref[...] == e, v1_ref[...], 0.0)
                 + jnp.where(i2_ref[...] == e, v2_ref[...], 0.0))
        acc = acc + scale * o

    @pl.when(step == 0)
    def _init():
        out_ref[...] = acc

    @pl.when(step != 0)
    def _acc():
        out_ref[...] += acc


def kernel(x, gw1, gb1, gw2, gb2, gw3, gb3, ew1, ew2, eb1, eb2):
    xt = x.reshape(T, DIM)
    eb1r = eb1.reshape(E, 1, H)
    eb2r = eb2.reshape(E, 1, DIM)

    i1, i2, v1, v2 = pl.pallas_call(
        _gate_kernel,
        out_shape=(
            jax.ShapeDtypeStruct((T, 1), jnp.int32),
            jax.ShapeDtypeStruct((T, 1), jnp.int32),
            jax.ShapeDtypeStruct((T, 1), jnp.float32),
            jax.ShapeDtypeStruct((T, 1), jnp.float32),
        ),
    )(xt, gw1, gb1.reshape(1, H), gw2, gb2.reshape(1, H), gw3,
      gb3.reshape(1, E))

    out = pl.pallas_call(
        _expert_kernel,
        grid=(E // EPB,),
        in_specs=[
            pl.BlockSpec((T, 1), lambda e: (0, 0)),
            pl.BlockSpec((T, 1), lambda e: (0, 0)),
            pl.BlockSpec((T, 1), lambda e: (0, 0)),
            pl.BlockSpec((T, 1), lambda e: (0, 0)),
            pl.BlockSpec((T, DIM), lambda e: (0, 0)),
            pl.BlockSpec((EPB, H, DIM), lambda e: (e, 0, 0)),
            pl.BlockSpec((EPB, H, DIM), lambda e: (e, 0, 0)),
            pl.BlockSpec((EPB, 1, H), lambda e: (e, 0, 0)),
            pl.BlockSpec((EPB, 1, DIM), lambda e: (e, 0, 0)),
        ],
        out_specs=pl.BlockSpec((T, DIM), lambda e: (0, 0)),
        out_shape=jax.ShapeDtypeStruct((T, DIM), jnp.float32),
        compiler_params=pltpu.CompilerParams(
            dimension_semantics=("arbitrary",),
        ),
    )(i1, i2, v1, v2, xt, ew1, ew2, eb1r, eb2r)

    return out.reshape(B, S, DIM)
